# baseline (speedup 1.0000x reference)
import functools

import jax
import jax.numpy as jnp
from jax.experimental import pallas as pl

_T, _D, _E, _K = 8192, 2048, 64, 8
_BT = 512


def _router_tile(h_ref, w_ref, mask_ref, probs_ref, logits_ref, logits_sel_ref):
    logits = jax.lax.dot_general(
        h_ref[...], w_ref[...],
        dimension_numbers=(((1,), (0,)), ((), ())),
        preferred_element_type=jnp.float32,
    )
    x = logits
    for _ in range(_K - 1):
        m = jnp.max(x, axis=-1, keepdims=True)
        x = jnp.where(x >= m, -jnp.inf, x)
    thr = jnp.max(x, axis=-1, keepdims=True)
    mask = logits >= thr
    rowmax = jnp.max(logits, axis=-1, keepdims=True)
    e = jnp.where(mask, jnp.exp(logits - rowmax), 0.0)
    probs = e / jnp.sum(e, axis=-1, keepdims=True)
    mask_ref[...] = mask
    probs_ref[...] = probs
    logits_ref[...] = logits
    logits_sel_ref[...] = logits


@jax.jit
def kernel(h, W):
    t, d = h.shape
    e = W.shape[1]
    grid = (t // _BT,)
    mask, probs, logits, logits_sel = pl.pallas_call(
        _router_tile,
        grid=grid,
        in_specs=[
            pl.BlockSpec((_BT, d), lambda i: (i, 0)),
            pl.BlockSpec((d, e), lambda i: (0, 0)),
        ],
        out_specs=[
            pl.BlockSpec((_BT, e), lambda i: (i, 0)),
            pl.BlockSpec((_BT, e), lambda i: (i, 0)),
            pl.BlockSpec((_BT, e), lambda i: (i, 0)),
            pl.BlockSpec((_BT, e), lambda i: (i, 0)),
        ],
        out_shape=[
            jax.ShapeDtypeStruct((t, e), jnp.bool_),
            jax.ShapeDtypeStruct((t, e), jnp.float32),
            jax.ShapeDtypeStruct((t, e), jnp.float32),
            jax.ShapeDtypeStruct((t, e), jnp.float32),
        ],
    )(h, W)
    return (mask, probs, logits, logits_sel)


# speedup vs baseline: 1.0154x; 1.0154x over previous
import functools

import jax
import jax.numpy as jnp
from jax.experimental import pallas as pl

_T, _D, _E, _K = 8192, 2048, 64, 8
_BT = 512


def _router_tile(h_ref, w_ref, mask_ref, probs_ref, logits_ref):
    logits = jax.lax.dot_general(
        h_ref[...], w_ref[...],
        dimension_numbers=(((1,), (0,)), ((), ())),
        preferred_element_type=jnp.float32,
    )
    x = logits
    for _ in range(_K - 1):
        m = jnp.max(x, axis=-1, keepdims=True)
        x = jnp.where(x >= m, -jnp.inf, x)
    thr = jnp.max(x, axis=-1, keepdims=True)
    mask = logits >= thr
    rowmax = jnp.max(logits, axis=-1, keepdims=True)
    e = jnp.where(mask, jnp.exp(logits - rowmax), 0.0)
    probs = e / jnp.sum(e, axis=-1, keepdims=True)
    mask_ref[...] = mask
    probs_ref[...] = probs
    logits_ref[...] = logits


@jax.jit
def kernel(h, W):
    t, d = h.shape
    e = W.shape[1]
    grid = (t // _BT,)
    mask, probs, logits = pl.pallas_call(
        _router_tile,
        grid=grid,
        in_specs=[
            pl.BlockSpec((_BT, d), lambda i: (i, 0)),
            pl.BlockSpec((d, e), lambda i: (0, 0)),
        ],
        out_specs=[
            pl.BlockSpec((_BT, e), lambda i: (i, 0)),
            pl.BlockSpec((_BT, e), lambda i: (i, 0)),
            pl.BlockSpec((_BT, e), lambda i: (i, 0)),
        ],
        out_shape=[
            jax.ShapeDtypeStruct((t, e), jnp.bool_),
            jax.ShapeDtypeStruct((t, e), jnp.float32),
            jax.ShapeDtypeStruct((t, e), jnp.float32),
        ],
    )(h, W)
    return (mask, probs, logits, logits)


# speedup vs baseline: 1.1650x; 1.1474x over previous
import functools

import jax
import jax.numpy as jnp
from jax.experimental import pallas as pl

_T, _D, _E, _K = 8192, 2048, 64, 8
_BT = 1024


def _router_tile(h_ref, w_ref, mask_ref, probs_ref, logits_ref):
    logits = jax.lax.dot_general(
        h_ref[...], w_ref[...],
        dimension_numbers=(((1,), (0,)), ((), ())),
        preferred_element_type=jnp.float32,
    )
    x = logits
    for _ in range(_K - 1):
        m = jnp.max(x, axis=-1, keepdims=True)
        x = jnp.where(x >= m, -jnp.inf, x)
    thr = jnp.max(x, axis=-1, keepdims=True)
    mask = logits >= thr
    rowmax = jnp.max(logits, axis=-1, keepdims=True)
    e = jnp.where(mask, jnp.exp(logits - rowmax), 0.0)
    probs = e / jnp.sum(e, axis=-1, keepdims=True)
    mask_ref[...] = mask.astype(jnp.int8)
    probs_ref[...] = probs
    logits_ref[...] = logits


@jax.jit
def kernel(h, W):
    t, d = h.shape
    e = W.shape[1]
    grid = (t // _BT,)
    mask, probs, logits = pl.pallas_call(
        _router_tile,
        grid=grid,
        in_specs=[
            pl.BlockSpec((_BT, d), lambda i: (i, 0)),
            pl.BlockSpec((d, e), lambda i: (0, 0)),
        ],
        out_specs=[
            pl.BlockSpec((_BT, e), lambda i: (i, 0)),
            pl.BlockSpec((_BT, e), lambda i: (i, 0)),
            pl.BlockSpec((_BT, e), lambda i: (i, 0)),
        ],
        out_shape=[
            jax.ShapeDtypeStruct((t, e), jnp.int8),
            jax.ShapeDtypeStruct((t, e), jnp.float32),
            jax.ShapeDtypeStruct((t, e), jnp.float32),
        ],
    )(h, W)
    return (mask.astype(bool), probs, logits, logits)


# speedup vs baseline: 1.1723x; 1.0062x over previous
import functools

import jax
import jax.numpy as jnp
from jax.experimental import pallas as pl

_T, _D, _E, _K = 8192, 2048, 64, 8
_BT = 2048


def _router_tile(h_ref, w_ref, mask_ref, probs_ref, logits_ref):
    logits = jax.lax.dot_general(
        h_ref[...], w_ref[...],
        dimension_numbers=(((1,), (0,)), ((), ())),
        preferred_element_type=jnp.float32,
    )
    x = logits
    for _ in range(_K - 1):
        m = jnp.max(x, axis=-1, keepdims=True)
        x = jnp.where(x >= m, -jnp.inf, x)
    thr = jnp.max(x, axis=-1, keepdims=True)
    mask = logits >= thr
    rowmax = jnp.max(logits, axis=-1, keepdims=True)
    e = jnp.where(mask, jnp.exp(logits - rowmax), 0.0)
    probs = e / jnp.sum(e, axis=-1, keepdims=True)
    mask_ref[...] = mask.astype(jnp.int8)
    probs_ref[...] = probs
    logits_ref[...] = logits


@jax.jit
def kernel(h, W):
    t, d = h.shape
    e = W.shape[1]
    grid = (t // _BT,)
    mask, probs, logits = pl.pallas_call(
        _router_tile,
        grid=grid,
        in_specs=[
            pl.BlockSpec((_BT, d), lambda i: (i, 0)),
            pl.BlockSpec((d, e), lambda i: (0, 0)),
        ],
        out_specs=[
            pl.BlockSpec((_BT, e), lambda i: (i, 0)),
            pl.BlockSpec((_BT, e), lambda i: (i, 0)),
            pl.BlockSpec((_BT, e), lambda i: (i, 0)),
        ],
        out_shape=[
            jax.ShapeDtypeStruct((t, e), jnp.int8),
            jax.ShapeDtypeStruct((t, e), jnp.float32),
            jax.ShapeDtypeStruct((t, e), jnp.float32),
        ],
    )(h, W)
    return (mask.astype(bool), probs, logits, logits)
